# Initial kernel scaffold; baseline (speedup 1.0000x reference)
import jax
import jax.numpy as jnp
from jax.experimental import pallas as pl


def kernel(seq1, seq2, edge_index, W, b, prelu_a, Wb, bb):
    raise NotImplementedError("write your pallas kernel here")


# speedup vs baseline: 9.8983x; 9.8983x over previous
import functools

import jax
import jax.numpy as jnp
from jax import lax
from jax.experimental import pallas as pl
from jax.experimental.pallas import tpu as pltpu
from jax.experimental.pallas import tpu_sc as plsc

N = 10000
D = 128
DH = D // 2
E = 320000
NP = 10240
EP = 327680
ER = EP // 128
ER_SC = ER // 2
ER_TILE_A = ER_SC // 16
ER_TILE_C = ER // 16
ROWS_TILE = NP // 16


def _deg_body(dst_hbm, z_hbm, ones_hbm, degs_hbm, dstv, onesv, acc):
    c = lax.axis_index("c")
    s = lax.axis_index("s")
    pltpu.sync_copy(z_hbm.at[pl.ds(s * ROWS_TILE, ROWS_TILE)],
                    acc.at[pl.ds(s * ROWS_TILE, ROWS_TILE)])
    pltpu.sync_copy(ones_hbm, onesv)
    base = c * ER_SC + s * ER_TILE_A
    pltpu.sync_copy(dst_hbm.at[pl.ds(base, ER_TILE_A)], dstv)
    plsc.subcore_barrier()

    def chunk(i, carry):
        pltpu.sync_copy(onesv, acc.at[dstv.at[i]], add=True)
        return carry

    lax.fori_loop(0, ER_TILE_A, chunk, 0)
    plsc.subcore_barrier()
    pltpu.sync_copy(acc.at[pl.ds(s * ROWS_TILE, ROWS_TILE)],
                    degs_hbm.at[pl.ds(c * NP + s * ROWS_TILE, ROWS_TILE)])


def _agg_body(ylo_hbm, yhi_hbm, srcg_hbm, dst_hbm, z_hbm, olo_hbm, ohi_hbm,
              srcv, dstv, rows, zbuf, acc):
    c = lax.axis_index("c")
    s = lax.axis_index("s")
    pltpu.sync_copy(z_hbm, zbuf)
    pltpu.sync_copy(srcg_hbm.at[c, pl.ds(s * ER_TILE_C, ER_TILE_C)], srcv)
    pltpu.sync_copy(dst_hbm.at[pl.ds(s * ER_TILE_C, ER_TILE_C)], dstv)

    for y_hbm, o_hbm in ((ylo_hbm, olo_hbm), (yhi_hbm, ohi_hbm)):
        for k in range(ROWS_TILE // 128):
            pltpu.sync_copy(zbuf, acc.at[pl.ds(s * ROWS_TILE + k * 128, 128)])
        plsc.subcore_barrier()

        def chunk(i, carry):
            pltpu.sync_copy(y_hbm.at[srcv.at[i]], rows)
            pltpu.sync_copy(rows, acc.at[dstv.at[i]], add=True)
            return carry

        lax.fori_loop(0, ER_TILE_C, chunk, 0)
        plsc.subcore_barrier()
        for k in range(ROWS_TILE // 128):
            r = s * ROWS_TILE + k * 128
            pltpu.sync_copy(acc.at[pl.ds(r, 128)],
                            o_hbm.at[pl.ds(c * NP + r, 128)])


def _b_body(x_ref, w_ref, d0_ref, d1_ref, ylo_ref, yhi_ref, dinvb_ref):
    deg = d0_ref[...] + d1_ref[...] + 1.0
    dinv = lax.rsqrt(deg)
    xw = jnp.dot(x_ref[...], w_ref[...], preferred_element_type=jnp.float32)
    y = xw * dinv
    ylo_ref[...] = y[:, :DH]
    yhi_ref[...] = y[:, DH:]
    dinvb_ref[...] = jnp.broadcast_to(dinv, y.shape)


def _d_body(alo_ref, ahi_ref, ylo_ref, yhi_ref, dv_ref, b_ref, pa_ref, o_ref):
    i = pl.program_id(0)

    @pl.when(i == 0)
    def _():
        o_ref[...] = jnp.zeros_like(o_ref)

    acc = jnp.concatenate([alo_ref[...] + ylo_ref[...],
                           ahi_ref[...] + yhi_ref[...]], axis=1)
    h = dv_ref[...] * acc + b_ref[...]
    a = pa_ref[0, 0]
    h = jnp.where(h >= 0, h, a * h)
    row = i * 1024 + lax.broadcasted_iota(jnp.int32, h.shape, 0)
    h = jnp.where(row < N, h, 0.0)
    o_ref[...] += jnp.sum(h, axis=0, keepdims=True)


def _e_body(alo1_ref, ahi1_ref, ylo1_ref, yhi1_ref, alo2_ref, ahi2_ref,
            ylo2_ref, yhi2_ref, dv_ref, ps_ref, wb_ref, b_ref, pa_ref, bb_ref,
            o1_ref, o2_ref):
    s = jax.nn.sigmoid(ps_ref[...] * (1.0 / N))
    v = lax.dot_general(wb_ref[...], s, (((1,), (1,)), ((), ())),
                        preferred_element_type=jnp.float32)
    a = pa_ref[0, 0]
    bias = b_ref[...]
    dv = dv_ref[...]

    def score(alo, ahi, ylo, yhi):
        acc = jnp.concatenate([alo[...] + ylo[...], ahi[...] + yhi[...]],
                              axis=1)
        h = dv * acc + bias
        h = jnp.where(h >= 0, h, a * h)
        return jnp.dot(h, v, preferred_element_type=jnp.float32) + bb_ref[0, 0]

    o1_ref[...] = score(alo1_ref, ahi1_ref, ylo1_ref, yhi1_ref)
    o2_ref[...] = score(alo2_ref, ahi2_ref, ylo2_ref, yhi2_ref)


def kernel(seq1, seq2, edge_index, W, b, prelu_a, Wb, bb):
    f32 = jnp.float32
    src = edge_index[0]
    dst = edge_index[1]
    pad_e = EP - E
    srcp = jnp.concatenate([src, jnp.zeros((pad_e,), jnp.int32)])
    dstp = jnp.concatenate([dst, jnp.full((pad_e,), N, jnp.int32)])
    dst2d = dstp.reshape(ER, 128)
    srcg = jnp.stack([srcp, srcp + NP]).reshape(2, ER, 128)
    seqs = jnp.zeros((2 * NP, D), f32)
    seqs = seqs.at[0:N].set(seq1).at[NP:NP + N].set(seq2)

    zeros1d = jnp.zeros((NP,), f32)
    ones128 = jnp.ones((128,), f32)
    zeros2d = jnp.zeros((128, DH), f32)

    mesh = plsc.VectorSubcoreMesh(core_axis_name="c", subcore_axis_name="s")

    deg_kernel = functools.partial(
        pl.kernel,
        out_type=jax.ShapeDtypeStruct((2 * NP,), f32),
        mesh=mesh,
        scratch_types=[
            pltpu.VMEM((ER_TILE_A, 128), jnp.int32),
            pltpu.VMEM((128,), f32),
            pltpu.VMEM_SHARED((NP,), f32),
        ],
    )(_deg_body)
    degs = deg_kernel(dst2d, zeros1d, ones128)

    deg0 = degs[:NP].reshape(NP, 1)
    deg1 = degs[NP:].reshape(NP, 1)

    nb = NP // 1024
    ylo, yhi, dinvb = pl.pallas_call(
        _b_body,
        grid=(2 * nb,),
        in_specs=[
            pl.BlockSpec((1024, D), lambda i: (i, 0)),
            pl.BlockSpec((D, D), lambda i: (0, 0)),
            pl.BlockSpec((1024, 1), lambda i: (i % nb, 0)),
            pl.BlockSpec((1024, 1), lambda i: (i % nb, 0)),
        ],
        out_specs=[
            pl.BlockSpec((1024, DH), lambda i: (i, 0)),
            pl.BlockSpec((1024, DH), lambda i: (i, 0)),
            pl.BlockSpec((1024, D), lambda i: (i % nb, 0)),
        ],
        out_shape=[
            jax.ShapeDtypeStruct((2 * NP, DH), f32),
            jax.ShapeDtypeStruct((2 * NP, DH), f32),
            jax.ShapeDtypeStruct((NP, D), f32),
        ],
    )(seqs, W, deg0, deg1)

    agg_kernel = functools.partial(
        pl.kernel,
        out_type=[
            jax.ShapeDtypeStruct((2 * NP, DH), f32),
            jax.ShapeDtypeStruct((2 * NP, DH), f32),
        ],
        mesh=mesh,
        scratch_types=[
            pltpu.VMEM((ER_TILE_C, 128), jnp.int32),
            pltpu.VMEM((ER_TILE_C, 128), jnp.int32),
            pltpu.VMEM((128, DH), f32),
            pltpu.VMEM((128, DH), f32),
            pltpu.VMEM_SHARED((NP, DH), f32),
        ],
        compiler_params=pltpu.CompilerParams(use_tc_tiling_on_sc=False),
    )(_agg_body)
    alo, ahi = agg_kernel(ylo, yhi, srcg, dst2d, zeros2d)

    b2d = b.reshape(1, D)
    pa2d = prelu_a.reshape(1, 1)
    bb2d = bb.reshape(1, 1)

    psum = pl.pallas_call(
        _d_body,
        grid=(nb,),
        in_specs=[
            pl.BlockSpec((1024, DH), lambda i: (i, 0)),
            pl.BlockSpec((1024, DH), lambda i: (i, 0)),
            pl.BlockSpec((1024, DH), lambda i: (i, 0)),
            pl.BlockSpec((1024, DH), lambda i: (i, 0)),
            pl.BlockSpec((1024, D), lambda i: (i, 0)),
            pl.BlockSpec((1, D), lambda i: (0, 0)),
            pl.BlockSpec((1, 1), lambda i: (0, 0)),
        ],
        out_specs=pl.BlockSpec((1, D), lambda i: (0, 0)),
        out_shape=jax.ShapeDtypeStruct((1, D), f32),
    )(alo, ahi, ylo, yhi, dinvb, b2d, pa2d)

    sc1, sc2 = pl.pallas_call(
        _e_body,
        grid=(nb,),
        in_specs=[
            pl.BlockSpec((1024, DH), lambda i: (i, 0)),
            pl.BlockSpec((1024, DH), lambda i: (i, 0)),
            pl.BlockSpec((1024, DH), lambda i: (i, 0)),
            pl.BlockSpec((1024, DH), lambda i: (i, 0)),
            pl.BlockSpec((1024, DH), lambda i: (i + nb, 0)),
            pl.BlockSpec((1024, DH), lambda i: (i + nb, 0)),
            pl.BlockSpec((1024, DH), lambda i: (i + nb, 0)),
            pl.BlockSpec((1024, DH), lambda i: (i + nb, 0)),
            pl.BlockSpec((1024, D), lambda i: (i, 0)),
            pl.BlockSpec((1, D), lambda i: (0, 0)),
            pl.BlockSpec((D, D), lambda i: (0, 0)),
            pl.BlockSpec((1, D), lambda i: (0, 0)),
            pl.BlockSpec((1, 1), lambda i: (0, 0)),
            pl.BlockSpec((1, 1), lambda i: (0, 0)),
        ],
        out_specs=[
            pl.BlockSpec((1024, 1), lambda i: (i, 0)),
            pl.BlockSpec((1024, 1), lambda i: (i, 0)),
        ],
        out_shape=[
            jax.ShapeDtypeStruct((NP, 1), f32),
            jax.ShapeDtypeStruct((NP, 1), f32),
        ],
    )(alo, ahi, ylo, yhi, alo, ahi, ylo, yhi, dinvb, psum, Wb[0], b2d, pa2d,
      bb2d)

    return jnp.concatenate([sc1[:N], sc2[:N]], axis=0)


# speedup vs baseline: 11.0672x; 1.1181x over previous
import functools

import jax
import jax.numpy as jnp
from jax import lax
from jax.experimental import pallas as pl
from jax.experimental.pallas import tpu as pltpu
from jax.experimental.pallas import tpu_sc as plsc

N = 10000
D = 128
DH = D // 2
E = 320000
NP = 10240
EP = 327680
ER = EP // 128
ER_SC = ER // 2
ER_TILE_A = ER_SC // 16
ER_TILE_C = ER // 16
ROWS_TILE = NP // 16


def _deg_body(dst_hbm, z_hbm, ones_hbm, degs_hbm, dstv, onesv, acc):
    c = lax.axis_index("c")
    s = lax.axis_index("s")
    pltpu.sync_copy(z_hbm.at[pl.ds(s * ROWS_TILE, ROWS_TILE)],
                    acc.at[pl.ds(s * ROWS_TILE, ROWS_TILE)])
    pltpu.sync_copy(ones_hbm, onesv)
    base = c * ER_SC + s * ER_TILE_A
    pltpu.sync_copy(dst_hbm.at[pl.ds(base, ER_TILE_A)], dstv)
    plsc.subcore_barrier()

    def chunk(i, carry):
        pltpu.sync_copy(onesv, acc.at[dstv.at[i]], add=True)
        return carry

    lax.fori_loop(0, ER_TILE_A, chunk, 0)
    plsc.subcore_barrier()
    pltpu.sync_copy(acc.at[pl.ds(s * ROWS_TILE, ROWS_TILE)],
                    degs_hbm.at[pl.ds(c * NP + s * ROWS_TILE, ROWS_TILE)])


def _agg_body(ylo_hbm, yhi_hbm, srcg_hbm, dst_hbm, z_hbm, olo_hbm, ohi_hbm,
              srcv, dstv, rows0, rows1, zbuf, acc, gsem0, gsem1):
    c = lax.axis_index("c")
    s = lax.axis_index("s")
    pltpu.sync_copy(z_hbm, zbuf)
    pltpu.sync_copy(srcg_hbm.at[c, pl.ds(s * ER_TILE_C, ER_TILE_C)], srcv)
    pltpu.sync_copy(dst_hbm.at[pl.ds(s * ER_TILE_C, ER_TILE_C)], dstv)

    for y_hbm, o_hbm in ((ylo_hbm, olo_hbm), (yhi_hbm, ohi_hbm)):
        for k in range(ROWS_TILE // 128):
            pltpu.sync_copy(zbuf, acc.at[pl.ds(s * ROWS_TILE + k * 128, 128)])
        plsc.subcore_barrier()

        pltpu.async_copy(y_hbm.at[srcv.at[0]], rows0, gsem0)

        def pair(j, carry):
            i0 = 2 * j
            pltpu.make_async_copy(y_hbm.at[srcv.at[i0]], rows0, gsem0).wait()
            pltpu.async_copy(y_hbm.at[srcv.at[i0 + 1]], rows1, gsem1)
            pltpu.sync_copy(rows0, acc.at[dstv.at[i0]], add=True)
            pltpu.make_async_copy(y_hbm.at[srcv.at[i0 + 1]], rows1,
                                  gsem1).wait()
            pltpu.async_copy(y_hbm.at[srcv.at[i0 + 2]], rows0, gsem0)
            pltpu.sync_copy(rows1, acc.at[dstv.at[i0 + 1]], add=True)
            return carry

        lax.fori_loop(0, ER_TILE_C // 2 - 1, pair, 0)
        iN = ER_TILE_C - 2
        pltpu.make_async_copy(y_hbm.at[srcv.at[iN]], rows0, gsem0).wait()
        pltpu.async_copy(y_hbm.at[srcv.at[iN + 1]], rows1, gsem1)
        pltpu.sync_copy(rows0, acc.at[dstv.at[iN]], add=True)
        pltpu.make_async_copy(y_hbm.at[srcv.at[iN + 1]], rows1, gsem1).wait()
        pltpu.sync_copy(rows1, acc.at[dstv.at[iN + 1]], add=True)

        plsc.subcore_barrier()
        for k in range(ROWS_TILE // 128):
            r = s * ROWS_TILE + k * 128
            pltpu.sync_copy(acc.at[pl.ds(r, 128)],
                            o_hbm.at[pl.ds(c * NP + r, 128)])


def _b_body(x_ref, w_ref, d0_ref, d1_ref, ylo_ref, yhi_ref, dinvb_ref):
    deg = d0_ref[...] + d1_ref[...] + 1.0
    dinv = lax.rsqrt(deg)
    xw = jnp.dot(x_ref[...], w_ref[...], preferred_element_type=jnp.float32)
    y = xw * dinv
    ylo_ref[...] = y[:, :DH]
    yhi_ref[...] = y[:, DH:]
    dinvb_ref[...] = jnp.broadcast_to(dinv, y.shape)


def _d_body(alo_ref, ahi_ref, ylo_ref, yhi_ref, dv_ref, b_ref, pa_ref, o_ref):
    i = pl.program_id(0)

    @pl.when(i == 0)
    def _():
        o_ref[...] = jnp.zeros_like(o_ref)

    acc = jnp.concatenate([alo_ref[...] + ylo_ref[...],
                           ahi_ref[...] + yhi_ref[...]], axis=1)
    h = dv_ref[...] * acc + b_ref[...]
    a = pa_ref[0, 0]
    h = jnp.where(h >= 0, h, a * h)
    row = i * 1024 + lax.broadcasted_iota(jnp.int32, h.shape, 0)
    h = jnp.where(row < N, h, 0.0)
    o_ref[...] += jnp.sum(h, axis=0, keepdims=True)


def _e_body(alo1_ref, ahi1_ref, ylo1_ref, yhi1_ref, alo2_ref, ahi2_ref,
            ylo2_ref, yhi2_ref, dv_ref, ps_ref, wb_ref, b_ref, pa_ref, bb_ref,
            o1_ref, o2_ref):
    s = jax.nn.sigmoid(ps_ref[...] * (1.0 / N))
    v = lax.dot_general(wb_ref[...], s, (((1,), (1,)), ((), ())),
                        preferred_element_type=jnp.float32)
    a = pa_ref[0, 0]
    bias = b_ref[...]
    dv = dv_ref[...]

    def score(alo, ahi, ylo, yhi):
        acc = jnp.concatenate([alo[...] + ylo[...], ahi[...] + yhi[...]],
                              axis=1)
        h = dv * acc + bias
        h = jnp.where(h >= 0, h, a * h)
        return jnp.dot(h, v, preferred_element_type=jnp.float32) + bb_ref[0, 0]

    o1_ref[...] = score(alo1_ref, ahi1_ref, ylo1_ref, yhi1_ref)
    o2_ref[...] = score(alo2_ref, ahi2_ref, ylo2_ref, yhi2_ref)


def kernel(seq1, seq2, edge_index, W, b, prelu_a, Wb, bb):
    f32 = jnp.float32
    src = edge_index[0]
    dst = edge_index[1]
    pad_e = EP - E
    srcp = jnp.concatenate([src, jnp.zeros((pad_e,), jnp.int32)])
    dstp = jnp.concatenate([dst, jnp.full((pad_e,), N, jnp.int32)])
    dst2d = dstp.reshape(ER, 128)
    srcg = jnp.stack([srcp, srcp + NP]).reshape(2, ER, 128)
    seqs = jnp.zeros((2 * NP, D), f32)
    seqs = seqs.at[0:N].set(seq1).at[NP:NP + N].set(seq2)

    zeros1d = jnp.zeros((NP,), f32)
    ones128 = jnp.ones((128,), f32)
    zeros2d = jnp.zeros((128, DH), f32)

    mesh = plsc.VectorSubcoreMesh(core_axis_name="c", subcore_axis_name="s")

    deg_kernel = functools.partial(
        pl.kernel,
        out_type=jax.ShapeDtypeStruct((2 * NP,), f32),
        mesh=mesh,
        scratch_types=[
            pltpu.VMEM((ER_TILE_A, 128), jnp.int32),
            pltpu.VMEM((128,), f32),
            pltpu.VMEM_SHARED((NP,), f32),
        ],
    )(_deg_body)
    degs = deg_kernel(dst2d, zeros1d, ones128)

    deg0 = degs[:NP].reshape(NP, 1)
    deg1 = degs[NP:].reshape(NP, 1)

    nb = NP // 1024
    ylo, yhi, dinvb = pl.pallas_call(
        _b_body,
        grid=(2 * nb,),
        in_specs=[
            pl.BlockSpec((1024, D), lambda i: (i, 0)),
            pl.BlockSpec((D, D), lambda i: (0, 0)),
            pl.BlockSpec((1024, 1), lambda i: (i % nb, 0)),
            pl.BlockSpec((1024, 1), lambda i: (i % nb, 0)),
        ],
        out_specs=[
            pl.BlockSpec((1024, DH), lambda i: (i, 0)),
            pl.BlockSpec((1024, DH), lambda i: (i, 0)),
            pl.BlockSpec((1024, D), lambda i: (i % nb, 0)),
        ],
        out_shape=[
            jax.ShapeDtypeStruct((2 * NP, DH), f32),
            jax.ShapeDtypeStruct((2 * NP, DH), f32),
            jax.ShapeDtypeStruct((NP, D), f32),
        ],
    )(seqs, W, deg0, deg1)

    agg_kernel = functools.partial(
        pl.kernel,
        out_type=[
            jax.ShapeDtypeStruct((2 * NP, DH), f32),
            jax.ShapeDtypeStruct((2 * NP, DH), f32),
        ],
        mesh=mesh,
        scratch_types=[
            pltpu.VMEM((ER_TILE_C, 128), jnp.int32),
            pltpu.VMEM((ER_TILE_C, 128), jnp.int32),
            pltpu.VMEM((128, DH), f32),
            pltpu.VMEM((128, DH), f32),
            pltpu.VMEM((128, DH), f32),
            pltpu.VMEM_SHARED((NP, DH), f32),
            pltpu.SemaphoreType.DMA,
            pltpu.SemaphoreType.DMA,
        ],
        compiler_params=pltpu.CompilerParams(use_tc_tiling_on_sc=False),
    )(_agg_body)
    alo, ahi = agg_kernel(ylo, yhi, srcg, dst2d, zeros2d)

    b2d = b.reshape(1, D)
    pa2d = prelu_a.reshape(1, 1)
    bb2d = bb.reshape(1, 1)

    psum = pl.pallas_call(
        _d_body,
        grid=(nb,),
        in_specs=[
            pl.BlockSpec((1024, DH), lambda i: (i, 0)),
            pl.BlockSpec((1024, DH), lambda i: (i, 0)),
            pl.BlockSpec((1024, DH), lambda i: (i, 0)),
            pl.BlockSpec((1024, DH), lambda i: (i, 0)),
            pl.BlockSpec((1024, D), lambda i: (i, 0)),
            pl.BlockSpec((1, D), lambda i: (0, 0)),
            pl.BlockSpec((1, 1), lambda i: (0, 0)),
        ],
        out_specs=pl.BlockSpec((1, D), lambda i: (0, 0)),
        out_shape=jax.ShapeDtypeStruct((1, D), f32),
    )(alo, ahi, ylo, yhi, dinvb, b2d, pa2d)

    sc1, sc2 = pl.pallas_call(
        _e_body,
        grid=(nb,),
        in_specs=[
            pl.BlockSpec((1024, DH), lambda i: (i, 0)),
            pl.BlockSpec((1024, DH), lambda i: (i, 0)),
            pl.BlockSpec((1024, DH), lambda i: (i, 0)),
            pl.BlockSpec((1024, DH), lambda i: (i, 0)),
            pl.BlockSpec((1024, DH), lambda i: (i + nb, 0)),
            pl.BlockSpec((1024, DH), lambda i: (i + nb, 0)),
            pl.BlockSpec((1024, DH), lambda i: (i + nb, 0)),
            pl.BlockSpec((1024, DH), lambda i: (i + nb, 0)),
            pl.BlockSpec((1024, D), lambda i: (i, 0)),
            pl.BlockSpec((1, D), lambda i: (0, 0)),
            pl.BlockSpec((D, D), lambda i: (0, 0)),
            pl.BlockSpec((1, D), lambda i: (0, 0)),
            pl.BlockSpec((1, 1), lambda i: (0, 0)),
            pl.BlockSpec((1, 1), lambda i: (0, 0)),
        ],
        out_specs=[
            pl.BlockSpec((1024, 1), lambda i: (i, 0)),
            pl.BlockSpec((1024, 1), lambda i: (i, 0)),
        ],
        out_shape=[
            jax.ShapeDtypeStruct((NP, 1), f32),
            jax.ShapeDtypeStruct((NP, 1), f32),
        ],
    )(alo, ahi, ylo, yhi, alo, ahi, ylo, yhi, dinvb, psum, Wb[0], b2d, pa2d,
      bb2d)

    return jnp.concatenate([sc1[:N], sc2[:N]], axis=0)
